# Initial kernel scaffold; baseline (speedup 1.0000x reference)
import jax
import jax.numpy as jnp
from jax.experimental import pallas as pl


def kernel(x, W):
    raise NotImplementedError("write your pallas kernel here")


# speedup vs baseline: 3.0206x; 3.0206x over previous
import functools

import jax
import jax.numpy as jnp
from jax import lax
from jax.experimental import pallas as pl
from jax.experimental.pallas import tpu as pltpu
from jax.experimental.pallas import tpu_sc as plsc

_HEADS = 4
_VOCAB = 100000
_DIM = 64
_B = 4096
_T = 200
_N = _B * _T * _HEADS
_NW = 32
_PER_W = _N // _NW
_GRP = 128
_K = 8
_STAGE = _GRP * _K
_NSTAGE = _PER_W // _STAGE


def _sc_body(idx_hbm, w_hbm, out_hbm, idx_v, rows_v, sem):
    wid = lax.axis_index("s") * 2 + lax.axis_index("c")
    base = wid * _PER_W
    pat = (lax.iota(jnp.int32, 16) & 3) * _VOCAB

    def stage(g, carry):
        row0 = base + g * _STAGE
        pltpu.sync_copy(idx_hbm.at[row0 // _STAGE], idx_v)
        for j in range(_K):
            for i in range(_GRP // 16):
                sl = (j, pl.ds(i * 16, 16))
                idx_v[sl] = idx_v[sl] + pat
        descs = [
            pltpu.async_copy(
                w_hbm.at[idx_v.at[j]],
                rows_v.at[pl.ds(j * _GRP, _GRP)],
                sem,
            )
            for j in range(_K)
        ]
        for d in descs:
            d.wait()
        pltpu.sync_copy(rows_v, out_hbm.at[pl.ds(row0, _STAGE)])
        return carry

    lax.fori_loop(0, _NSTAGE, stage, 0)


@jax.jit
def kernel(x, W):
    idx = x.reshape(_N // _STAGE, _K, _GRP)
    w2 = W.reshape(_HEADS * _VOCAB, _DIM)
    mesh = plsc.VectorSubcoreMesh(core_axis_name="c", subcore_axis_name="s")
    out = pl.kernel(
        _sc_body,
        mesh=mesh,
        compiler_params=pltpu.CompilerParams(use_tc_tiling_on_sc=False),
        out_type=jax.ShapeDtypeStruct((_N, _DIM), jnp.float32),
        scratch_types=[
            pltpu.VMEM((_K, _GRP), jnp.int32),
            pltpu.VMEM((_STAGE, _DIM), jnp.float32),
            pltpu.SemaphoreType.DMA,
        ],
    )(idx, w2)
    return out.reshape(_B, _T, _HEADS, _DIM)
